# Initial kernel scaffold; baseline (speedup 1.0000x reference)
import jax
import jax.numpy as jnp
from jax.experimental import pallas as pl


def kernel(x, edge_index, edge_att, W1, b1, W2, b2, temp):
    raise NotImplementedError("write your pallas kernel here")


# speedup vs baseline: 6.0656x; 6.0656x over previous
import functools
import math

import numpy as np
import jax
import jax.numpy as jnp
from jax import lax
from jax.experimental import pallas as pl
from jax.experimental.pallas import tpu as pltpu
from jax.experimental.pallas import tpu_sc as plsc

K = 10
N = 10000
E = 320000
F = 128

NC = 2
NS = 16
NW = NC * NS
KB = 128
CH = 79
EPW = CH * KB
EPAD = NW * EPW

NPAD = 10240
RPW = NPAD // NS
BLK = 640
GRID = NPAD // BLK


def _cheb_scalar(i, x):
    if i == 0:
        return 1.0
    if i == 1:
        return x
    t0, t1 = 1.0, x
    for _ in range(2, i + 1):
        t0, t1 = t1, 2.0 * x * t1 - t0
    return t1


def _coe_rows():
    rows = np.zeros((K + 1, 128), dtype=np.float32)
    for i in range(K + 1):
        for j in range(K + 1):
            xj = math.cos((K - j + 0.5) * math.pi / (K + 1))
            rows[i, j] = _cheb_scalar(i, xj) * (2.0 / (K + 1))
    return rows


_CROWS = _coe_rows()


def _coe_dot(i, temp_ref):
    ci = jnp.float32(0.0)
    for j in range(K + 1):
        ci = ci + float(_CROWS[i, j]) * temp_ref[0, j]
    return ci


_sc_mesh = plsc.VectorSubcoreMesh(core_axis_name="c", subcore_axis_name="s")


@functools.partial(
    pl.kernel,
    out_type=jax.ShapeDtypeStruct((NC, NPAD, F), jnp.float32),
    mesh=_sc_mesh,
    scratch_types=[
        pltpu.VMEM((CH, KB), jnp.int32),
        pltpu.VMEM((CH, KB), jnp.int32),
        pltpu.VMEM((KB, F), jnp.float32),
        pltpu.VMEM_SHARED((NPAD, F), jnp.float32),
        pltpu.SemaphoreType.DMA,
    ],
)
def _sc_prop(table_hbm, srcs_hbm, dsts_hbm, out_hbm, src_v, dst_v, rows_v,
             acc, sem):
    c = lax.axis_index("c")
    s = lax.axis_index("s")
    w = s * NC + c

    def _zrow(i, carry):
        for q in range(F // 16):
            rows_v[i, pl.ds(q * 16, 16)] = jnp.zeros((16,), jnp.float32)
        return carry

    lax.fori_loop(0, KB, _zrow, 0)
    base = s * RPW
    for k in range(RPW // KB):
        pltpu.sync_copy(rows_v, acc.at[pl.ds(base + k * KB, KB)])

    pltpu.sync_copy(srcs_hbm.at[w], src_v)
    pltpu.sync_copy(dsts_hbm.at[w], dst_v)
    plsc.subcore_barrier()

    def _chunk(j, carry):
        pltpu.async_copy(table_hbm.at[src_v.at[j]], rows_v, sem).wait()
        pltpu.sync_copy(rows_v, acc.at[dst_v.at[j]], add=True)
        return carry

    lax.fori_loop(0, CH, _chunk, 0)
    plsc.subcore_barrier()

    for k in range(RPW // KB):
        sl = pl.ds(base + k * KB, KB)
        pltpu.sync_copy(acc.at[sl], out_hbm.at[c].at[sl])


def _mlp_body(x_ref, w1_ref, b1_ref, w2_ref, b2_ref, deg2_ref, temp_ref,
              h_ref, u_ref, out0_ref, dis_ref):
    b = pl.program_id(0)
    xb = x_ref[...]
    hb = lax.dot_general(xb, w1_ref[...], (((1,), (1,)), ((), ())),
                         preferred_element_type=jnp.float32) + b1_ref[...]
    hb = jnp.maximum(hb, 0.0)
    hb = lax.dot_general(hb, w2_ref[...], (((1,), (1,)), ((), ())),
                         preferred_element_type=jnp.float32) + b2_ref[...]
    rowid = b * BLK + lax.broadcasted_iota(jnp.int32, (BLK, 1), 0)
    hb = jnp.where(rowid < N, hb, 0.0)
    deg = deg2_ref[0, :, 0:1] + deg2_ref[1, :, 0:1]
    pos = deg > 0.0
    dis = jnp.where(pos, lax.rsqrt(jnp.where(pos, deg, 1.0)), 0.0)
    coe0 = _coe_dot(0, temp_ref)
    h_ref[...] = hb
    u_ref[...] = dis * hb
    out0_ref[...] = (0.5 * coe0) * hb
    dis_ref[...] = dis


_mlp = pl.pallas_call(
    _mlp_body,
    grid=(GRID,),
    in_specs=[
        pl.BlockSpec((BLK, F), lambda b: (b, 0)),
        pl.BlockSpec((F, F), lambda b: (0, 0)),
        pl.BlockSpec((1, F), lambda b: (0, 0)),
        pl.BlockSpec((F, F), lambda b: (0, 0)),
        pl.BlockSpec((1, F), lambda b: (0, 0)),
        pl.BlockSpec((NC, BLK, F), lambda b: (0, b, 0)),
        pl.BlockSpec((1, 128), lambda b: (0, 0)),
    ],
    out_specs=[
        pl.BlockSpec((BLK, F), lambda b: (b, 0)),
        pl.BlockSpec((BLK, F), lambda b: (b, 0)),
        pl.BlockSpec((BLK, F), lambda b: (b, 0)),
        pl.BlockSpec((BLK, 1), lambda b: (b, 0)),
    ],
    out_shape=[
        jax.ShapeDtypeStruct((NPAD, F), jnp.float32),
        jax.ShapeDtypeStruct((NPAD, F), jnp.float32),
        jax.ShapeDtypeStruct((NPAD, F), jnp.float32),
        jax.ShapeDtypeStruct((NPAD, 1), jnp.float32),
    ],
)


def _step_body(s_ref, dis_ref, tp_ref, oin_ref, temp_ref,
               tx_ref, out_ref, u_ref, *, alpha, beta, i):
    sblk = s_ref[0] + s_ref[1]
    d = dis_ref[...]
    ci = _coe_dot(i, temp_ref)
    t = alpha * (d * sblk) + beta * tp_ref[...]
    tx_ref[...] = t
    out_ref[...] = oin_ref[...] + ci * t
    u_ref[...] = d * t


def _make_step(i):
    alpha = -1.0 if i == 1 else -2.0
    beta = 0.0 if i == 1 else -1.0
    return pl.pallas_call(
        functools.partial(_step_body, alpha=alpha, beta=beta, i=i),
        grid=(GRID,),
        in_specs=[
            pl.BlockSpec((NC, BLK, F), lambda b: (0, b, 0)),
            pl.BlockSpec((BLK, 1), lambda b: (b, 0)),
            pl.BlockSpec((BLK, F), lambda b: (b, 0)),
            pl.BlockSpec((BLK, F), lambda b: (b, 0)),
            pl.BlockSpec((1, 128), lambda b: (0, 0)),
        ],
        out_specs=[
            pl.BlockSpec((BLK, F), lambda b: (b, 0)),
            pl.BlockSpec((BLK, F), lambda b: (b, 0)),
            pl.BlockSpec((BLK, F), lambda b: (b, 0)),
        ],
        out_shape=[
            jax.ShapeDtypeStruct((NPAD, F), jnp.float32),
            jax.ShapeDtypeStruct((NPAD, F), jnp.float32),
            jax.ShapeDtypeStruct((NPAD, F), jnp.float32),
        ],
    )


_steps = {i: _make_step(i) for i in range(1, K + 1)}


def kernel(x, edge_index, edge_att, W1, b1, W2, b2, temp):
    del edge_att
    row = edge_index[0]
    col = edge_index[1]
    fill = jnp.full((EPAD - E,), N, jnp.int32)
    srcs = jnp.concatenate([row, fill]).reshape(NW, CH, KB)
    dsts = jnp.concatenate([col, fill]).reshape(NW, CH, KB)
    x_pad = jnp.pad(x, ((0, NPAD - N), (0, 0)))
    temp_pad = jnp.zeros((1, 128), jnp.float32).at[0, : K + 1].set(temp)
    ones_tab = jnp.ones((NPAD, F), jnp.float32)

    deg2 = _sc_prop(ones_tab, srcs, srcs)
    h, u, out, dis = _mlp(x_pad, W1, b1.reshape(1, F), W2, b2.reshape(1, F),
                          deg2, temp_pad)
    tx0 = h
    s_p = _sc_prop(u, srcs, dsts)
    tx1, out, u = _steps[1](s_p, dis, tx0, out, temp_pad)
    for i in range(2, K + 1):
        s_p = _sc_prop(u, srcs, dsts)
        tx2, out, u = _steps[i](s_p, dis, tx1, out, temp_pad)
        tx0, tx1 = tx1, tx2
    return out[:N]
